# Initial kernel scaffold; baseline (speedup 1.0000x reference)
import jax
import jax.numpy as jnp
from jax.experimental import pallas as pl


def kernel(attn_s):
    raise NotImplementedError("write your pallas kernel here")


# speedup vs baseline: 17.9660x; 17.9660x over previous
import jax
import jax.numpy as jnp
from jax.experimental import pallas as pl

_K = 32
_EPS = 1e-7


def _body(x_ref, o_ref):
    x = x_ref[...]
    bits = jax.lax.bitcast_convert_type(x, jnp.int32)

    def step(i, cur):
        cand = cur | (jnp.int32(1) << (jnp.int32(30) - i))
        cnt = jnp.sum((bits >= cand).astype(jnp.int32), axis=1, keepdims=True)
        return jnp.where(cnt >= _K, cand, cur)

    kth_bits = jax.lax.fori_loop(0, 31, step, jnp.zeros((x.shape[0], 1), jnp.int32))
    kth = jax.lax.bitcast_convert_type(kth_bits, jnp.float32)
    delta = kth + _EPS
    w = jnp.maximum(x - delta, 0.0)
    s = jnp.sum(w, axis=1, keepdims=True) + _EPS
    o_ref[...] = w / s


def kernel(attn_s):
    return pl.pallas_call(
        _body,
        out_shape=jax.ShapeDtypeStruct(attn_s.shape, attn_s.dtype),
    )(attn_s)
